# baseline (speedup 1.0000x reference)
import functools

import jax
import jax.numpy as jnp
from jax import lax
from jax.experimental import pallas as pl
from jax.experimental.pallas import tpu as pltpu
from jax.experimental.pallas import tpu_sc as plsc

MOMENTUM = 0.9
M, D, B = 100000, 512, 16384

NC, NS = 2, 16
NW = NC * NS
B_PER_W = B // NW
CH = 64
NCHUNK = B_PER_W // CH

B1 = 12288
B2 = B - B1

_sc_mesh = plsc.VectorSubcoreMesh(core_axis_name="c", subcore_axis_name="s")


@functools.partial(
    pl.kernel,
    mesh=_sc_mesh,
    out_type=jax.ShapeDtypeStruct((B, D), jnp.float32),
    scratch_types=[
        pltpu.VMEM((CH,), jnp.int32),
        pltpu.VMEM((CH, D), jnp.float32),
        pltpu.SemaphoreType.DMA,
    ],
)
def _sc_gather(mem_hbm, idx_hbm, old_hbm, idx_v, rows_v, sem):
    wid = lax.axis_index("s") * NC + lax.axis_index("c")
    base = wid * B_PER_W

    def body(c, _):
        off = base + c * CH
        pltpu.sync_copy(idx_hbm.at[pl.ds(off, CH)], idx_v)
        pltpu.async_copy(mem_hbm.at[idx_v], rows_v, sem).wait()
        pltpu.sync_copy(rows_v, old_hbm.at[pl.ds(off, CH)])
        return 0

    lax.fori_loop(0, NCHUNK, body, 0)


def _make_scatter(nrows, gbase):
    bpw = nrows // NW
    nch = bpw // CH

    @functools.partial(
        pl.kernel,
        mesh=_sc_mesh,
        out_type=(),
        scratch_types=[
            pltpu.VMEM((CH,), jnp.int32),
            pltpu.VMEM((CH, D), jnp.float32),
            pltpu.SemaphoreType.DMA,
        ],
    )
    def _k(normed_hbm, idx_hbm, out_ref, idx_v, rows_v, sem):
        wid = lax.axis_index("s") * NC + lax.axis_index("c")

        def body(c, _):
            loc = wid * bpw + c * CH
            pltpu.sync_copy(idx_hbm.at[pl.ds(gbase + loc, CH)], idx_v)
            pltpu.sync_copy(normed_hbm.at[pl.ds(loc, CH)], rows_v)
            pltpu.async_copy(rows_v, out_ref.at[idx_v], sem).wait()
            return 0

        lax.fori_loop(0, nch, body, 0)

    return _k


_sc_scatter1 = _make_scatter(B1, 0)
_sc_scatter2 = _make_scatter(B2, B1)


MC = 3128
PRE = 8
GRID1 = 32 - PRE
BM = B1 // GRID1
GRID2 = B2 // BM


def _copy_body(mem_ref, copy_ref):
    copy_ref[...] = mem_ref[...]


_tc_precopy = pl.pallas_call(
    _copy_body,
    grid=(PRE,),
    in_specs=[pl.BlockSpec((MC, D), lambda i: (i, 0))],
    out_specs=pl.BlockSpec((MC, D), lambda i: (i, 0)),
    out_shape=jax.ShapeDtypeStruct((M, D), jnp.float32),
)


def _mlp_norm(val_blk, w1, b1, w2, b2, old_blk):
    h = jnp.tanh(
        jnp.dot(val_blk, w1, preferred_element_type=jnp.float32) + b1
    )
    v = jnp.dot(h, w2, preferred_element_type=jnp.float32) + b2
    blended = MOMENTUM * old_blk + (1.0 - MOMENTUM) * v
    ss = jnp.sum(blended * blended, axis=1, keepdims=True)
    return blended / (jnp.sqrt(ss) + 1e-8)


def _tc_body1(val_ref, w1_ref, b1_ref, w2_ref, b2_ref, old_ref, mem_ref,
              out_in_ref, normed_ref, copy_ref):
    normed_ref[...] = _mlp_norm(
        val_ref[...], w1_ref[...], b1_ref[...], w2_ref[...], b2_ref[...],
        old_ref[...],
    )
    copy_ref[...] = mem_ref[...]


_tc_fused1 = pl.pallas_call(
    _tc_body1,
    grid=(GRID1,),
    in_specs=[
        pl.BlockSpec((BM, D), lambda i: (i, 0)),
        pl.BlockSpec((D, D), lambda i: (0, 0)),
        pl.BlockSpec((1, D), lambda i: (0, 0)),
        pl.BlockSpec((D, D), lambda i: (0, 0)),
        pl.BlockSpec((1, D), lambda i: (0, 0)),
        pl.BlockSpec((BM, D), lambda i: (i, 0)),
        pl.BlockSpec((MC, D), lambda i: (i + PRE, 0)),
        pl.BlockSpec(memory_space=pl.ANY),
    ],
    out_specs=[
        pl.BlockSpec((BM, D), lambda i: (i, 0)),
        pl.BlockSpec((MC, D), lambda i: (i + PRE, 0)),
    ],
    out_shape=[
        jax.ShapeDtypeStruct((B1, D), jnp.float32),
        jax.ShapeDtypeStruct((M, D), jnp.float32),
    ],
    input_output_aliases={7: 1},
)


def _tc_body2(val_ref, w1_ref, b1_ref, w2_ref, b2_ref, old_ref, normed_ref):
    normed_ref[...] = _mlp_norm(
        val_ref[...], w1_ref[...], b1_ref[...], w2_ref[...], b2_ref[...],
        old_ref[...],
    )


_tc_fused2 = pl.pallas_call(
    _tc_body2,
    grid=(GRID2,),
    in_specs=[
        pl.BlockSpec((BM, D), lambda i: (i + GRID1, 0)),
        pl.BlockSpec((D, D), lambda i: (0, 0)),
        pl.BlockSpec((1, D), lambda i: (0, 0)),
        pl.BlockSpec((D, D), lambda i: (0, 0)),
        pl.BlockSpec((1, D), lambda i: (0, 0)),
        pl.BlockSpec((BM, D), lambda i: (i + GRID1, 0)),
    ],
    out_specs=pl.BlockSpec((BM, D), lambda i: (i, 0)),
    out_shape=jax.ShapeDtypeStruct((B2, D), jnp.float32),
)


def kernel(mem, val, W1, b1, W2, b2, idx):
    b1r = b1.reshape(1, D)
    b2r = b2.reshape(1, D)
    old = _sc_gather(mem, idx)
    out1 = _tc_precopy(mem)
    normed1, out2 = _tc_fused1(val, W1, b1r, W2, b2r, old, mem, out1)
    out_ref = jax.new_ref(out2)
    _sc_scatter1(normed1, idx, out_ref)
    normed2 = _tc_fused2(val, W1, b1r, W2, b2r, old)
    _sc_scatter2(normed2, idx, out_ref)
    return out_ref[...]


# speedup vs baseline: 1.0558x; 1.0558x over previous
import functools

import jax
import jax.numpy as jnp
from jax import lax
from jax.experimental import pallas as pl
from jax.experimental.pallas import tpu as pltpu
from jax.experimental.pallas import tpu_sc as plsc

MOMENTUM = 0.9
M, D, B = 100000, 512, 16384

NC, NS = 2, 16
NW = NC * NS
B_PER_W = B // NW
CH = 64
NCHUNK = B_PER_W // CH

_sc_mesh = plsc.VectorSubcoreMesh(core_axis_name="c", subcore_axis_name="s")


@functools.partial(
    pl.kernel,
    mesh=_sc_mesh,
    out_type=jax.ShapeDtypeStruct((B, D), jnp.float32),
    scratch_types=[
        pltpu.VMEM((CH,), jnp.int32),
        pltpu.VMEM((CH, D), jnp.float32),
        pltpu.SemaphoreType.DMA,
    ],
)
def _sc_gather(mem_hbm, idx_hbm, old_hbm, idx_v, rows_v, sem):
    wid = lax.axis_index("s") * NC + lax.axis_index("c")
    base = wid * B_PER_W

    def body(c, _):
        off = base + c * CH
        pltpu.sync_copy(idx_hbm.at[pl.ds(off, CH)], idx_v)
        pltpu.async_copy(mem_hbm.at[idx_v], rows_v, sem).wait()
        pltpu.sync_copy(rows_v, old_hbm.at[pl.ds(off, CH)])
        return 0

    lax.fori_loop(0, NCHUNK, body, 0)


@functools.partial(
    pl.kernel,
    mesh=_sc_mesh,
    out_type=(),
    scratch_types=[
        pltpu.VMEM((CH,), jnp.int32),
        pltpu.VMEM((CH,), jnp.int32),
        pltpu.VMEM((CH, D), jnp.float32),
        pltpu.VMEM((CH, D), jnp.float32),
        pltpu.SemaphoreType.DMA,
        pltpu.SemaphoreType.DMA,
    ],
)
def _sc_scatter(normed_hbm, idx_hbm, out_ref, idx_a, idx_b, rows_a, rows_b,
                lsem, ssem):
    wid = lax.axis_index("s") * NC + lax.axis_index("c")
    base = wid * B_PER_W
    idxb = [idx_a, idx_b]
    rowsb = [rows_a, rows_b]

    def start_loads(c):
        off = base + c * CH
        return (
            pltpu.async_copy(idx_hbm.at[pl.ds(off, CH)], idxb[c % 2], lsem),
            pltpu.async_copy(normed_hbm.at[pl.ds(off, CH)], rowsb[c % 2], lsem),
        )

    scat = {}
    loads = start_loads(0)
    for c in range(NCHUNK):
        nxt = None
        if c + 1 < NCHUNK:
            if c >= 1:
                scat[c - 1].wait()
            nxt = start_loads(c + 1)
        for h in loads:
            h.wait()
        scat[c] = pltpu.async_copy(rowsb[c % 2], out_ref.at[idxb[c % 2]], ssem)
        if nxt is not None:
            loads = nxt
    scat[NCHUNK - 2].wait()
    scat[NCHUNK - 1].wait()


MC = 3128
PRE = 8
GRID = 32 - PRE
BM = 688


def _copy_body(mem_ref, copy_ref):
    copy_ref[...] = mem_ref[...]


_tc_precopy = pl.pallas_call(
    _copy_body,
    grid=(PRE,),
    in_specs=[pl.BlockSpec((MC, D), lambda i: (i, 0))],
    out_specs=pl.BlockSpec((MC, D), lambda i: (i, 0)),
    out_shape=jax.ShapeDtypeStruct((M, D), jnp.float32),
)


def _tc_body(val_ref, w1_ref, b1_ref, w2_ref, b2_ref, old_ref, mem_ref,
             out_in_ref, normed_ref, copy_ref):
    h = jnp.tanh(
        jnp.dot(val_ref[...], w1_ref[...], preferred_element_type=jnp.float32)
        + b1_ref[...]
    )
    v = (
        jnp.dot(h, w2_ref[...], preferred_element_type=jnp.float32)
        + b2_ref[...]
    )
    blended = MOMENTUM * old_ref[...] + (1.0 - MOMENTUM) * v
    ss = jnp.sum(blended * blended, axis=1, keepdims=True)
    normed_ref[...] = blended / (jnp.sqrt(ss) + 1e-8)
    copy_ref[...] = mem_ref[...]


_tc_fused = pl.pallas_call(
    _tc_body,
    grid=(GRID,),
    in_specs=[
        pl.BlockSpec((BM, D), lambda i: (i, 0)),
        pl.BlockSpec((D, D), lambda i: (0, 0)),
        pl.BlockSpec((1, D), lambda i: (0, 0)),
        pl.BlockSpec((D, D), lambda i: (0, 0)),
        pl.BlockSpec((1, D), lambda i: (0, 0)),
        pl.BlockSpec((BM, D), lambda i: (i, 0)),
        pl.BlockSpec((MC, D), lambda i: (i + PRE, 0)),
        pl.BlockSpec(memory_space=pl.ANY),
    ],
    out_specs=[
        pl.BlockSpec((BM, D), lambda i: (i, 0)),
        pl.BlockSpec((MC, D), lambda i: (i + PRE, 0)),
    ],
    out_shape=[
        jax.ShapeDtypeStruct((B, D), jnp.float32),
        jax.ShapeDtypeStruct((M, D), jnp.float32),
    ],
    input_output_aliases={7: 1},
)


def kernel(mem, val, W1, b1, W2, b2, idx):
    old = _sc_gather(mem, idx)
    out1 = _tc_precopy(mem)
    normed, out = _tc_fused(
        val, W1, b1.reshape(1, D), W2, b2.reshape(1, D), old, mem, out1
    )
    out_ref = jax.new_ref(out)
    _sc_scatter(normed, idx, out_ref)
    return out_ref[...]
